# baseline (speedup 1.0000x reference)
import jax
import jax.numpy as jnp
from jax import lax
from jax.experimental import pallas as pl
from jax.experimental.pallas import tpu as pltpu
from jax.experimental.pallas import tpu_sc as plsc

_NU = 50000
_NI = 50000
_N = _NU + _NI
_H = 16
_E = 32
_NL = 3
_NE = 1600000
_NCE = 20000
_NCP = 1024

_f32 = jnp.float32
_i32 = jnp.int32

_SC_PARAMS = pltpu.CompilerParams(needs_layout_passes=False,
                                  use_tc_tiling_on_sc=False)
_MESH = plsc.VectorSubcoreMesh(core_axis_name="c", subcore_axis_name="s")


def _make_gcn(acc_rows, rows_per_tile, zrows, nzcopy,
              groups_per_tile, chunks):
    out_type = tuple(jax.ShapeDtypeStruct((2 * acc_rows, _H), _f32)
                     for _ in range(_NL))
    scratch = [
        pltpu.VMEM_SHARED((acc_rows, _H), _f32),
        pltpu.VMEM((2, 4, 2, 128), _i32),
        pltpu.VMEM((2, 512), _f32),
        pltpu.VMEM((2, 4, 128), _i32),
        pltpu.VMEM((2, 512, _H), _f32),
        pltpu.VMEM((zrows, _H), _f32),
        pltpu.SemaphoreType.DMA,
        pltpu.SemaphoreType.DMA,
        pltpu.SemaphoreType.DMA,
        pltpu.SemaphoreType.DMA,
    ]

    def body(cr_hbm, val_hbm, x_hbm, o1, o2, o3,
             acc, crb, valb, gidx, gath, zbuf, gs0, gs1, ss0, ss1):
        c = lax.axis_index("c")
        s = lax.axis_index("s")
        base_rows = s * rows_per_tile
        coff = c * acc_rows
        gsems = (gs0, gs1)
        ssems = (ss0, ss1)

        def _zb(i, _):
            zbuf[i, :] = jnp.zeros((_H,), _f32)
            return 0
        lax.fori_loop(0, zrows, _zb, 0)

        def _run_layer(src, dst):
            def _zero(k, _):
                pltpu.sync_copy(zbuf,
                                acc.at[pl.ds(base_rows + k * zrows, zrows), :])
                return 0
            lax.fori_loop(0, nzcopy, _zero, 0)
            plsc.subcore_barrier()

            def _load_fire(b, ci):
                g0 = s * groups_per_tile + ci * 4
                pltpu.sync_copy(cr_hbm.at[pl.ds(g0, 4), :, :], crb.at[b])
                pltpu.sync_copy(val_hbm.at[pl.ds(g0 * 128, 512)], valb.at[b])

                def _gx(g, _):
                    for j in range(8):
                        gidx[b, g, pl.ds(j * 16, 16)] = (
                            crb[b, g, 0, pl.ds(j * 16, 16)] + coff)
                    return 0
                lax.fori_loop(0, 4, _gx, 0)
                for gi in range(4):
                    pltpu.make_async_copy(
                        src.at[gidx.at[b, gi]],
                        gath.at[b, pl.ds(gi * 128, 128), :],
                        gsems[b]).start()

            def _drain_gather(b):
                for gi in range(4):
                    pltpu.make_async_copy(
                        src.at[gidx.at[b, gi]],
                        gath.at[b, pl.ds(gi * 128, 128), :],
                        gsems[b]).wait()

            def _scale(b):
                @plsc.parallel_loop(0, 32, unroll=4)
                def _(j):
                    t0 = j * 16
                    tvec = jnp.broadcast_to(t0, (16,)).astype(_i32)
                    for i in range(16):
                        vv = plsc.load_gather(valb.at[b], [tvec + i])
                        gath[b, t0 + i, :] = gath[b, t0 + i, :] * vv

            def _fire_scatter(b):
                for gi in range(4):
                    pltpu.make_async_copy(
                        gath.at[b, pl.ds(gi * 128, 128), :],
                        acc.at[crb.at[b, gi, 1]], ssems[b]).start(add=True)

            def _drain_scatter(b):
                for gi in range(4):
                    pltpu.make_async_copy(
                        gath.at[b, pl.ds(gi * 128, 128), :],
                        acc.at[crb.at[b, gi, 1]], ssems[b]).wait()

            half = chunks // 2
            _load_fire(0, 0)

            def _pair(p, _):
                c0 = p * 2

                @pl.when(p > 0)
                def _():
                    _drain_scatter(1)
                _load_fire(1, c0 + 1)
                _drain_gather(0)
                _scale(0)
                _fire_scatter(0)

                @pl.when(p < half - 1)
                def _():
                    _drain_scatter(0)
                    _load_fire(0, c0 + 2)
                _drain_gather(1)
                _scale(1)
                _fire_scatter(1)
                return 0
            lax.fori_loop(0, half, _pair, 0)
            _drain_scatter(0)
            _drain_scatter(1)
            plsc.subcore_barrier()

            def _wb(k, _):
                sl = pl.ds(base_rows + k * zrows, zrows)
                pltpu.sync_copy(acc.at[sl, :],
                                dst.at[pl.ds(coff + base_rows + k * zrows,
                                             zrows), :])
                return 0
            lax.fori_loop(0, nzcopy, _wb, 0)
            plsc.subcore_barrier()

        srcs = [x_hbm, o1, o2]
        dsts = [o1, o2, o3]
        for layer in range(_NL):
            _run_layer(srcs[layer], dsts[layer])

    return pl.kernel(body, out_type=out_type, mesh=_MESH,
                     scratch_types=scratch, compiler_params=_SC_PARAMS)


_gcn_main = _make_gcn(acc_rows=_N, rows_per_tile=6250, zrows=125,
                      nzcopy=50, groups_per_tile=784, chunks=196)
_EP_MAIN = 16 * 784 * 128

_gcn_com = _make_gcn(acc_rows=_NCP, rows_per_tile=64, zrows=64,
                     nzcopy=1, groups_per_tile=16, chunks=4)
_EP_COM = 16 * 16 * 128


def _make_com_mean():
    out_type = jax.ShapeDtypeStruct((2 * _NCP, _H), _f32)
    scratch = [pltpu.VMEM((4, 64, _H), _f32), pltpu.SemaphoreType.DMA]

    def body(x0, o1, o2, o3, out, mbuf, sem):
        c = lax.axis_index("c")
        s = lax.axis_index("s")
        row0 = c * _NCP + s * 64
        tbls = [x0, o1, o2, o3]
        for t in range(4):
            pltpu.sync_copy(tbls[t].at[pl.ds(row0, 64), :], mbuf.at[t])

        def _mrow(r, _):
            m = (mbuf[0, r, :] + mbuf[1, r, :]
                 + mbuf[2, r, :] + mbuf[3, r, :]) * 0.25
            mbuf[0, r, :] = m
            return 0
        lax.fori_loop(0, 64, _mrow, 0)
        pltpu.sync_copy(mbuf.at[0], out.at[pl.ds(row0, 64), :])

    return pl.kernel(body, out_type=out_type, mesh=_MESH,
                     scratch_types=scratch, compiler_params=_SC_PARAMS)


_com_mean = _make_com_mean()


def _make_final_gather():
    out_type = (
        jax.ShapeDtypeStruct((2 * 4096, _H), _f32),
        jax.ShapeDtypeStruct((2 * 4096, _H), _f32),
        jax.ShapeDtypeStruct((2 * 4096, _H), _f32),
        jax.ShapeDtypeStruct((4096, _E), _f32),
        jax.ShapeDtypeStruct((4096, _E), _f32),
        jax.ShapeDtypeStruct((4096, _E), _f32),
    )
    scratch = [
        pltpu.VMEM((1, 128), _i32),
        pltpu.VMEM((1, 128), _i32),
        pltpu.VMEM((4, 128, _H), _f32),
        pltpu.VMEM((128, _E), _f32),
        pltpu.SemaphoreType.DMA,
    ]

    def body(x0, o1, o2, o3, e3u, e3i, users, items, negs,
             mu, mi, mn, pu, pi_, pn, idb, gidx, gbuf, gbuf32, sem):
        c = lax.axis_index("c")
        s = lax.axis_index("s")
        wid = s * 2 + c
        tbls = [x0, o1, o2, o3]
        for ids, roff, out in ((users, 0, mu), (items, _NU, mi),
                               (negs, _NU, mn)):
            for g in range(2):
                grp = s * 2 + g
                pltpu.sync_copy(ids.at[pl.ds(grp, 1), :], idb)
                off = c * _N + roff

                def _gx(j, _):
                    gidx[0, pl.ds(j * 16, 16)] = (
                        idb[0, pl.ds(j * 16, 16)] + off)
                    return 0
                lax.fori_loop(0, 8, _gx, 0)
                descs = [pltpu.make_async_copy(tbls[t].at[gidx.at[0]],
                                               gbuf.at[t], sem)
                         for t in range(4)]
                for d in descs:
                    d.start()
                for d in descs:
                    d.wait()

                def _mrow(r, _):
                    m = (gbuf[0, r, :] + gbuf[1, r, :]
                         + gbuf[2, r, :] + gbuf[3, r, :]) * 0.25
                    gbuf[0, r, :] = m
                    return 0
                lax.fori_loop(0, 128, _mrow, 0)
                pltpu.sync_copy(
                    gbuf.at[0],
                    out.at[pl.ds(c * 4096 + grp * 128, 128), :])
        for tbl, ids, out in ((e3u, users, pu), (e3i, items, pi_),
                              (e3i, negs, pn)):
            pltpu.sync_copy(ids.at[pl.ds(wid, 1), :], idb)
            pltpu.make_async_copy(tbl.at[idb.at[0]], gbuf32, sem).start()
            pltpu.make_async_copy(tbl.at[idb.at[0]], gbuf32, sem).wait()
            pltpu.sync_copy(gbuf32, out.at[pl.ds(wid * 128, 128), :])

    return pl.kernel(body, out_type=out_type, mesh=_MESH,
                     scratch_types=scratch, compiler_params=_SC_PARAMS)


_gatherk = _make_final_gather()


def _matT_body(a_ref, b_ref, o_ref, acc_ref):
    k = pl.program_id(0)

    @pl.when(k == 0)
    def _():
        acc_ref[...] = jnp.zeros_like(acc_ref)

    acc_ref[...] += lax.dot_general(a_ref[...], b_ref[...],
                                    (((0,), (0,)), ((), ())),
                                    preferred_element_type=_f32)

    @pl.when(k == pl.num_programs(0) - 1)
    def _():
        o_ref[...] = acc_ref[...]


def _matT(a, b):
    K, C = b.shape
    kb = 2000
    return pl.pallas_call(
        _matT_body,
        grid=(K // kb,),
        in_specs=[pl.BlockSpec((kb, a.shape[1]), lambda k: (k, 0)),
                  pl.BlockSpec((kb, C), lambda k: (k, 0))],
        out_specs=pl.BlockSpec((a.shape[1], C), lambda k: (0, 0)),
        out_shape=jax.ShapeDtypeStruct((a.shape[1], C), _f32),
        scratch_shapes=[pltpu.VMEM((a.shape[1], C), _f32)],
    )(a, b)


def _mm_body(a_ref, b_ref, o_ref):
    o_ref[...] = jnp.dot(a_ref[...], b_ref[...], preferred_element_type=_f32)


def _mm(a, b):
    M, K = a.shape
    C = b.shape[1]
    mb = 2000
    return pl.pallas_call(
        _mm_body,
        grid=(M // mb,),
        in_specs=[pl.BlockSpec((mb, K), lambda k: (k, 0)),
                  pl.BlockSpec((K, C), lambda k: (0, 0))],
        out_specs=pl.BlockSpec((mb, C), lambda k: (k, 0)),
        out_shape=jax.ShapeDtypeStruct((M, C), _f32),
    )(a, b)


def _dots_body(mu, mi, mn, pu, pi_, pn, o):
    o[...] = (jnp.sum(mu[...] * (mi[...] - mn[...]), axis=1, keepdims=True)
              + jnp.sum(pu[...] * (pi_[...] - pn[...]), axis=1, keepdims=True))


def _dots(mu, mi, mn, pu, pi_, pn):
    return pl.pallas_call(
        _dots_body,
        out_shape=jax.ShapeDtypeStruct((4096, 1), _f32),
    )(mu, mi, mn, pu, pi_, pn)


def _half_stack(x):
    return jnp.concatenate([x[:, :_H], x[:, _H:]], axis=0)


def _half_unstack(x):
    r = x.shape[0] // 2
    return jnp.concatenate([x[:r], x[r:]], axis=1)


def kernel(adj_indices, adj_values, uc, ic, com_indices, com_values,
           users, items, neg_items, user_emb, item_emb):
    ego = jnp.concatenate([user_emb, item_emb], axis=0)
    x0 = _half_stack(ego)
    rowp = jnp.pad(adj_indices[0], (0, _EP_MAIN - _NE)).reshape(-1, 128)
    colp = jnp.pad(adj_indices[1], (0, _EP_MAIN - _NE)).reshape(-1, 128)
    crp = jnp.stack([colp, rowp], axis=1)
    valp = jnp.pad(adj_values, (0, _EP_MAIN - _NE))
    o1, o2, o3 = _gcn_main(crp, valp, x0)

    cu = _matT(user_emb, uc)
    cit = _matT(item_emb, ic)
    ego2 = jnp.concatenate([cu.T, cit.T], axis=0)
    ego2p = jnp.pad(ego2, ((0, _NCP - 1000), (0, 0)))
    x0c = _half_stack(ego2p)
    crow = jnp.pad(com_indices[0], (0, _EP_COM - _NCE)).reshape(-1, 128)
    ccol = jnp.pad(com_indices[1], (0, _EP_COM - _NCE)).reshape(-1, 128)
    ccrp = jnp.stack([ccol, crow], axis=1)
    cval = jnp.pad(com_values, (0, _EP_COM - _NCE))
    c1, c2, c3 = _gcn_com(ccrp, cval, x0c)
    allcom = _com_mean(x0c, c1, c2, c3)
    acf = _half_unstack(allcom)

    e3u = _mm(uc, acf[0:500])
    e3i = _mm(ic, acf[500:1000])

    u2 = users.reshape(32, 128)
    i2 = items.reshape(32, 128)
    n2 = neg_items.reshape(32, 128)
    mu, mi, mn, pu, pi_, pn = _gatherk(x0, o1, o2, o3, e3u, e3i, u2, i2, n2)

    logits = _dots(_half_unstack(mu), _half_unstack(mi), _half_unstack(mn),
                   pu, pi_, pn)
    return logits[:, 0]


# speedup vs baseline: 1.0558x; 1.0558x over previous
import jax
import jax.numpy as jnp
from jax import lax
from jax.experimental import pallas as pl
from jax.experimental.pallas import tpu as pltpu
from jax.experimental.pallas import tpu_sc as plsc

_NU = 50000
_NI = 50000
_N = _NU + _NI
_H = 16
_E = 32
_NL = 3
_NE = 1600000
_NCE = 20000
_NCP = 1024

_f32 = jnp.float32
_i32 = jnp.int32

_SC_PARAMS = pltpu.CompilerParams(needs_layout_passes=False,
                                  use_tc_tiling_on_sc=False)
_MESH = plsc.VectorSubcoreMesh(core_axis_name="c", subcore_axis_name="s")


def _make_gcn(acc_rows, rows_per_tile, zrows, nzcopy,
              groups_per_tile, chunks):
    out_type = tuple(jax.ShapeDtypeStruct((2 * acc_rows, _H), _f32)
                     for _ in range(_NL))
    scratch = [
        pltpu.VMEM_SHARED((acc_rows, _H), _f32),
        pltpu.VMEM((2, 4, 2, 128), _i32),
        pltpu.VMEM((2, 512), _f32),
        pltpu.VMEM((2, 4, 128), _i32),
        pltpu.VMEM((2, 512, _H), _f32),
        pltpu.VMEM((zrows, _H), _f32),
        pltpu.SemaphoreType.DMA,
        pltpu.SemaphoreType.DMA,
        pltpu.SemaphoreType.DMA,
        pltpu.SemaphoreType.DMA,
        pltpu.SemaphoreType.DMA,
        pltpu.SemaphoreType.DMA,
    ]

    def body(cr_hbm, val_hbm, x_hbm, o1, o2, o3,
             acc, crb, valb, gidx, gath, zbuf, gs0, gs1, ss0, ss1,
             is0, is1):
        c = lax.axis_index("c")
        s = lax.axis_index("s")
        base_rows = s * rows_per_tile
        coff = c * acc_rows
        gsems = (gs0, gs1)
        ssems = (ss0, ss1)
        isems = (is0, is1)

        def _zb(i, _):
            zbuf[i, :] = jnp.zeros((_H,), _f32)
            return 0
        lax.fori_loop(0, zrows, _zb, 0)

        def _run_layer(src, dst):
            def _zero(k, _):
                pltpu.sync_copy(zbuf,
                                acc.at[pl.ds(base_rows + k * zrows, zrows), :])
                return 0
            lax.fori_loop(0, nzcopy, _zero, 0)
            plsc.subcore_barrier()

            def _start_input(b, ci):
                g0 = s * groups_per_tile + ci * 4
                pltpu.make_async_copy(cr_hbm.at[pl.ds(g0, 4), :, :],
                                      crb.at[b], isems[b]).start()
                pltpu.make_async_copy(val_hbm.at[pl.ds(g0 * 128, 512)],
                                      valb.at[b], isems[b]).start()

            def _wait_input_fire(b, ci):
                g0 = s * groups_per_tile + ci * 4
                pltpu.make_async_copy(cr_hbm.at[pl.ds(g0, 4), :, :],
                                      crb.at[b], isems[b]).wait()
                pltpu.make_async_copy(val_hbm.at[pl.ds(g0 * 128, 512)],
                                      valb.at[b], isems[b]).wait()

                def _gx(g, _):
                    for j in range(8):
                        gidx[b, g, pl.ds(j * 16, 16)] = (
                            crb[b, g, 0, pl.ds(j * 16, 16)] + coff)
                    return 0
                lax.fori_loop(0, 4, _gx, 0)
                for gi in range(4):
                    pltpu.make_async_copy(
                        src.at[gidx.at[b, gi]],
                        gath.at[b, pl.ds(gi * 128, 128), :],
                        gsems[b]).start()

            def _drain_gather(b):
                for gi in range(4):
                    pltpu.make_async_copy(
                        src.at[gidx.at[b, gi]],
                        gath.at[b, pl.ds(gi * 128, 128), :],
                        gsems[b]).wait()

            def _scale(b):
                @plsc.parallel_loop(0, 32, unroll=4)
                def _(j):
                    t0 = j * 16
                    tvec = jnp.broadcast_to(t0, (16,)).astype(_i32)
                    for i in range(16):
                        vv = plsc.load_gather(valb.at[b], [tvec + i])
                        gath[b, t0 + i, :] = gath[b, t0 + i, :] * vv

            def _fire_scatter(b):
                for gi in range(4):
                    pltpu.make_async_copy(
                        gath.at[b, pl.ds(gi * 128, 128), :],
                        acc.at[crb.at[b, gi, 1]], ssems[b]).start(add=True)

            def _drain_scatter(b):
                for gi in range(4):
                    pltpu.make_async_copy(
                        gath.at[b, pl.ds(gi * 128, 128), :],
                        acc.at[crb.at[b, gi, 1]], ssems[b]).wait()

            half = chunks // 2
            _start_input(0, 0)
            _wait_input_fire(0, 0)

            def _pair(p, _):
                c0 = p * 2

                @pl.when(p > 0)
                def _():
                    _drain_scatter(1)
                _start_input(1, c0 + 1)
                _drain_gather(0)
                _scale(0)
                _fire_scatter(0)
                _wait_input_fire(1, c0 + 1)

                @pl.when(p < half - 1)
                def _():
                    _drain_scatter(0)
                    _start_input(0, c0 + 2)
                _drain_gather(1)
                _scale(1)
                _fire_scatter(1)

                @pl.when(p < half - 1)
                def _():
                    _wait_input_fire(0, c0 + 2)
                return 0
            lax.fori_loop(0, half, _pair, 0)
            _drain_scatter(0)
            _drain_scatter(1)
            plsc.subcore_barrier()

            def _wb(k, _):
                sl = pl.ds(base_rows + k * zrows, zrows)
                pltpu.sync_copy(acc.at[sl, :],
                                dst.at[pl.ds(coff + base_rows + k * zrows,
                                             zrows), :])
                return 0
            lax.fori_loop(0, nzcopy, _wb, 0)
            plsc.subcore_barrier()

        srcs = [x_hbm, o1, o2]
        dsts = [o1, o2, o3]
        for layer in range(_NL):
            _run_layer(srcs[layer], dsts[layer])

    return pl.kernel(body, out_type=out_type, mesh=_MESH,
                     scratch_types=scratch, compiler_params=_SC_PARAMS)


_gcn_main = _make_gcn(acc_rows=_N, rows_per_tile=6250, zrows=125,
                      nzcopy=50, groups_per_tile=784, chunks=196)
_EP_MAIN = 16 * 784 * 128

_EP_COM = 16 * 16 * 128


def _make_com_gcn():
    out_type = tuple(jax.ShapeDtypeStruct((2 * _NCP, _H), _f32)
                     for _ in range(4))
    scratch = [
        pltpu.VMEM_SHARED((_NCP, _H), _f32),
        pltpu.VMEM((16, 2, 128), _i32),
        pltpu.VMEM((2048,), _f32),
        pltpu.VMEM((16, 128), _i32),
        pltpu.VMEM((2048, _H), _f32),
        pltpu.VMEM((64, _H), _f32),
        pltpu.VMEM((4, 64, _H), _f32),
        pltpu.SemaphoreType.DMA,
    ]

    def body(cr_hbm, val_hbm, x_hbm, o1, o2, o3, om,
             acc, crb, valb, gidx, gath, zbuf, mbuf, sem):
        c = lax.axis_index("c")
        s = lax.axis_index("s")
        coff = c * _NCP
        base_rows = s * 64

        def _zb(i, _):
            zbuf[i, :] = jnp.zeros((_H,), _f32)
            return 0
        lax.fori_loop(0, 64, _zb, 0)

        g0 = s * 16
        pltpu.sync_copy(cr_hbm.at[pl.ds(g0, 16), :, :], crb)
        pltpu.sync_copy(val_hbm.at[pl.ds(g0 * 128, 2048)], valb)

        def _gx(g, _):
            for j in range(8):
                gidx[g, pl.ds(j * 16, 16)] = (
                    crb[g, 0, pl.ds(j * 16, 16)] + coff)
            return 0
        lax.fori_loop(0, 16, _gx, 0)

        srcs = [x_hbm, o1, o2]
        dsts = [o1, o2, o3]
        for layer in range(_NL):
            src, dst = srcs[layer], dsts[layer]
            pltpu.sync_copy(zbuf, acc.at[pl.ds(base_rows, 64), :])
            plsc.subcore_barrier()
            for gi in range(16):
                pltpu.make_async_copy(
                    src.at[gidx.at[gi]],
                    gath.at[pl.ds(gi * 128, 128), :], sem).start()
            for gi in range(16):
                pltpu.make_async_copy(
                    src.at[gidx.at[gi]],
                    gath.at[pl.ds(gi * 128, 128), :], sem).wait()

            @plsc.parallel_loop(0, 128, unroll=4)
            def _(j):
                t0 = j * 16
                tvec = jnp.broadcast_to(t0, (16,)).astype(_i32)
                for i in range(16):
                    vv = plsc.load_gather(valb, [tvec + i])
                    gath[t0 + i, :] = gath[t0 + i, :] * vv

            for gi in range(16):
                pltpu.sync_copy(gath.at[pl.ds(gi * 128, 128), :],
                                acc.at[crb.at[gi, 1]], add=True)
            plsc.subcore_barrier()
            pltpu.sync_copy(acc.at[pl.ds(base_rows, 64), :],
                            dst.at[pl.ds(coff + base_rows, 64), :])
            plsc.subcore_barrier()

        tbls = [x_hbm, o1, o2, o3]
        row0 = coff + base_rows
        for t in range(4):
            pltpu.sync_copy(tbls[t].at[pl.ds(row0, 64), :], mbuf.at[t])

        def _mrow(r, _):
            m = (mbuf[0, r, :] + mbuf[1, r, :]
                 + mbuf[2, r, :] + mbuf[3, r, :]) * 0.25
            mbuf[0, r, :] = m
            return 0
        lax.fori_loop(0, 64, _mrow, 0)
        pltpu.sync_copy(mbuf.at[0], om.at[pl.ds(row0, 64), :])

    return pl.kernel(body, out_type=out_type, mesh=_MESH,
                     scratch_types=scratch, compiler_params=_SC_PARAMS)


_gcn_com = _make_com_gcn()


def _make_final_gather():
    out_type = (
        jax.ShapeDtypeStruct((2 * 4096, _H), _f32),
        jax.ShapeDtypeStruct((2 * 4096, _H), _f32),
        jax.ShapeDtypeStruct((2 * 4096, _H), _f32),
        jax.ShapeDtypeStruct((4096, _E), _f32),
        jax.ShapeDtypeStruct((4096, _E), _f32),
        jax.ShapeDtypeStruct((4096, _E), _f32),
    )
    scratch = [
        pltpu.VMEM((1, 128), _i32),
        pltpu.VMEM((1, 128), _i32),
        pltpu.VMEM((4, 128, _H), _f32),
        pltpu.VMEM((128, _E), _f32),
        pltpu.SemaphoreType.DMA,
    ]

    def body(x0, o1, o2, o3, e3u, e3i, users, items, negs,
             mu, mi, mn, pu, pi_, pn, idb, gidx, gbuf, gbuf32, sem):
        c = lax.axis_index("c")
        s = lax.axis_index("s")
        wid = s * 2 + c
        tbls = [x0, o1, o2, o3]
        for ids, roff, out in ((users, 0, mu), (items, _NU, mi),
                               (negs, _NU, mn)):
            for g in range(2):
                grp = s * 2 + g
                pltpu.sync_copy(ids.at[pl.ds(grp, 1), :], idb)
                off = c * _N + roff

                def _gx(j, _):
                    gidx[0, pl.ds(j * 16, 16)] = (
                        idb[0, pl.ds(j * 16, 16)] + off)
                    return 0
                lax.fori_loop(0, 8, _gx, 0)
                descs = [pltpu.make_async_copy(tbls[t].at[gidx.at[0]],
                                               gbuf.at[t], sem)
                         for t in range(4)]
                for d in descs:
                    d.start()
                for d in descs:
                    d.wait()

                def _mrow(r, _):
                    m = (gbuf[0, r, :] + gbuf[1, r, :]
                         + gbuf[2, r, :] + gbuf[3, r, :]) * 0.25
                    gbuf[0, r, :] = m
                    return 0
                lax.fori_loop(0, 128, _mrow, 0)
                pltpu.sync_copy(
                    gbuf.at[0],
                    out.at[pl.ds(c * 4096 + grp * 128, 128), :])
        for tbl, ids, out in ((e3u, users, pu), (e3i, items, pi_),
                              (e3i, negs, pn)):
            pltpu.sync_copy(ids.at[pl.ds(wid, 1), :], idb)
            pltpu.make_async_copy(tbl.at[idb.at[0]], gbuf32, sem).start()
            pltpu.make_async_copy(tbl.at[idb.at[0]], gbuf32, sem).wait()
            pltpu.sync_copy(gbuf32, out.at[pl.ds(wid * 128, 128), :])

    return pl.kernel(body, out_type=out_type, mesh=_MESH,
                     scratch_types=scratch, compiler_params=_SC_PARAMS)


_gatherk = _make_final_gather()


def _matT_body(a_ref, b_ref, o_ref, acc_ref):
    k = pl.program_id(0)

    @pl.when(k == 0)
    def _():
        acc_ref[...] = jnp.zeros_like(acc_ref)

    acc_ref[...] += lax.dot_general(a_ref[...], b_ref[...],
                                    (((0,), (0,)), ((), ())),
                                    preferred_element_type=_f32)

    @pl.when(k == pl.num_programs(0) - 1)
    def _():
        o_ref[...] = acc_ref[...]


def _matT(a, b):
    K, C = b.shape
    kb = 2000
    return pl.pallas_call(
        _matT_body,
        grid=(K // kb,),
        in_specs=[pl.BlockSpec((kb, a.shape[1]), lambda k: (k, 0)),
                  pl.BlockSpec((kb, C), lambda k: (k, 0))],
        out_specs=pl.BlockSpec((a.shape[1], C), lambda k: (0, 0)),
        out_shape=jax.ShapeDtypeStruct((a.shape[1], C), _f32),
        scratch_shapes=[pltpu.VMEM((a.shape[1], C), _f32)],
    )(a, b)


def _mm_body(a_ref, b_ref, o_ref):
    o_ref[...] = jnp.dot(a_ref[...], b_ref[...], preferred_element_type=_f32)


def _mm(a, b):
    M, K = a.shape
    C = b.shape[1]
    mb = 2000
    return pl.pallas_call(
        _mm_body,
        grid=(M // mb,),
        in_specs=[pl.BlockSpec((mb, K), lambda k: (k, 0)),
                  pl.BlockSpec((K, C), lambda k: (0, 0))],
        out_specs=pl.BlockSpec((mb, C), lambda k: (k, 0)),
        out_shape=jax.ShapeDtypeStruct((M, C), _f32),
    )(a, b)


def _dots_body(mu, mi, mn, pu, pi_, pn, o):
    o[...] = (jnp.sum(mu[...] * (mi[...] - mn[...]), axis=1, keepdims=True)
              + jnp.sum(pu[...] * (pi_[...] - pn[...]), axis=1, keepdims=True))


def _dots(mu, mi, mn, pu, pi_, pn):
    return pl.pallas_call(
        _dots_body,
        out_shape=jax.ShapeDtypeStruct((4096, 1), _f32),
    )(mu, mi, mn, pu, pi_, pn)


def _half_stack(x):
    return jnp.concatenate([x[:, :_H], x[:, _H:]], axis=0)


def _half_unstack(x):
    r = x.shape[0] // 2
    return jnp.concatenate([x[:r], x[r:]], axis=1)


def kernel(adj_indices, adj_values, uc, ic, com_indices, com_values,
           users, items, neg_items, user_emb, item_emb):
    ego = jnp.concatenate([user_emb, item_emb], axis=0)
    x0 = _half_stack(ego)
    rowp = jnp.pad(adj_indices[0], (0, _EP_MAIN - _NE)).reshape(-1, 128)
    colp = jnp.pad(adj_indices[1], (0, _EP_MAIN - _NE)).reshape(-1, 128)
    crp = jnp.stack([colp, rowp], axis=1)
    valp = jnp.pad(adj_values, (0, _EP_MAIN - _NE))
    o1, o2, o3 = _gcn_main(crp, valp, x0)

    cu = _matT(user_emb, uc)
    cit = _matT(item_emb, ic)
    ego2 = jnp.concatenate([cu.T, cit.T], axis=0)
    ego2p = jnp.pad(ego2, ((0, _NCP - 1000), (0, 0)))
    x0c = _half_stack(ego2p)
    crow = jnp.pad(com_indices[0], (0, _EP_COM - _NCE)).reshape(-1, 128)
    ccol = jnp.pad(com_indices[1], (0, _EP_COM - _NCE)).reshape(-1, 128)
    ccrp = jnp.stack([ccol, crow], axis=1)
    cval = jnp.pad(com_values, (0, _EP_COM - _NCE))
    _, _, _, allcom = _gcn_com(ccrp, cval, x0c)
    acf = _half_unstack(allcom)

    e3u = _mm(uc, acf[0:500])
    e3i = _mm(ic, acf[500:1000])

    u2 = users.reshape(32, 128)
    i2 = items.reshape(32, 128)
    n2 = neg_items.reshape(32, 128)
    mu, mi, mn, pu, pi_, pn = _gatherk(x0, o1, o2, o3, e3u, e3i, u2, i2, n2)

    logits = _dots(_half_unstack(mu), _half_unstack(mi), _half_unstack(mn),
                   pu, pi_, pn)
    return logits[:, 0]
